# baseline (speedup 1.0000x reference)
import functools

import jax
import jax.numpy as jnp
from jax import lax
from jax.experimental import pallas as pl
from jax.experimental.pallas import tpu as pltpu
from jax.experimental.pallas import tpu_sc as plsc

NUM_ROWS = 16384
NUM_COLS = 26
DIM = 32

B_TOTAL = NUM_ROWS * NUM_COLS
NUM_EMB = 1000000
NC = 2
NS = 16
NW = NC * NS
B_PER_W = B_TOTAL // NW
CHUNK = 128
N_CHUNKS = B_PER_W // CHUNK
NBUF = 4
N_GROUPS = N_CHUNKS // NBUF
G2 = N_GROUPS // 2

SB = 512
N_SLABS = NUM_EMB // SB
SLABS_PER_W = 61
TAIL0 = N_SLABS * SB
TAIL = NUM_EMB - TAIL0

_mesh = plsc.VectorSubcoreMesh(core_axis_name="c", subcore_axis_name="s")


@functools.partial(
    pl.kernel,
    out_type=jax.ShapeDtypeStruct((NUM_EMB * DIM // 128, 128), jnp.float32),
    mesh=_mesh,
    scratch_types=[
        pltpu.VMEM((DIM, SB), jnp.float32),
        pltpu.VMEM((DIM, TAIL), jnp.float32),
        pltpu.VMEM((SB // 4, 128), jnp.float32),
        pltpu.SemaphoreType.DMA,
    ],
    compiler_params=pltpu.CompilerParams(use_tc_tiling_on_sc=True,
                                         needs_layout_passes=False),
)
def _transpose_table(wt_hbm, out_hbm, slab, tail_slab, ot, sem):
    wid = lax.axis_index("s") * NC + lax.axis_index("c")

    def transpose_slab(width, src, carry=0):
        def per_g(g, c2):
            v = g * 16 + lax.iota(jnp.int32, 16)
            row = v >> 2
            col0 = (v & 3) * DIM
            for d in range(DIM):
                plsc.store_scatter(ot, [row, col0 + d],
                                   src[d, pl.ds(g * 16, 16)])
            return c2

        return lax.fori_loop(0, width // 16, per_g, carry)

    def do_slab(k, carry):
        v0 = pl.multiple_of(k * SB, SB)
        pltpu.async_copy(wt_hbm.at[:, pl.ds(v0, SB)], slab, sem).wait()
        transpose_slab(SB, slab)
        pltpu.async_copy(
            ot, out_hbm.at[pl.ds(pl.multiple_of(v0 // 4, SB // 4), SB // 4)],
            sem).wait()
        return carry

    lax.fori_loop(wid * SLABS_PER_W, (wid + 1) * SLABS_PER_W, do_slab, 0)

    @pl.when(wid == 0)
    def _():
        do_slab(SLABS_PER_W * NW, 0)
        pltpu.async_copy(wt_hbm.at[:, pl.ds(TAIL0, TAIL)], tail_slab,
                         sem).wait()
        transpose_slab(TAIL, tail_slab)
        pltpu.async_copy(ot.at[pl.ds(0, TAIL // 4)],
                         out_hbm.at[pl.ds(TAIL0 // 4, TAIL // 4)],
                         sem).wait()


@functools.partial(
    pl.kernel,
    out_type=jax.ShapeDtypeStruct((B_TOTAL, DIM), jnp.float32),
    mesh=_mesh,
    scratch_types=[
        pltpu.VMEM((N_CHUNKS, CHUNK), jnp.int32),
        pltpu.VMEM((NBUF, CHUNK, DIM), jnp.float32),
        pltpu.VMEM((NBUF, CHUNK, DIM), jnp.float32),
        pltpu.SemaphoreType.DMA,
        pltpu.SemaphoreType.DMA,
        pltpu.SemaphoreType.DMA,
        pltpu.SemaphoreType.DMA,
    ],
    compiler_params=pltpu.CompilerParams(use_tc_tiling_on_sc=False),
)
def _emb_lookup(idx_hbm, table_hbm, out_hbm, idx_v, buf_a, buf_b,
                gsem_a, gsem_b, ssem_a, ssem_b):
    wid = lax.axis_index("s") * NC + lax.axis_index("c")
    base = wid * B_PER_W
    pltpu.sync_copy(idx_hbm.at[wid], idx_v)

    def fire_gathers(g, buf, gsem):
        for b in range(NBUF):
            i = g * NBUF + b
            pltpu.async_copy(table_hbm.at[idx_v.at[i]], buf.at[b], gsem)

    def drain_gathers(g, buf, gsem):
        for b in range(NBUF):
            i = g * NBUF + b
            pltpu.make_async_copy(table_hbm.at[idx_v.at[i]], buf.at[b],
                                  gsem).wait()

    def fire_stores(g, buf, ssem):
        for b in range(NBUF):
            i = g * NBUF + b
            pltpu.async_copy(buf.at[b],
                             out_hbm.at[pl.ds(base + i * CHUNK, CHUNK)], ssem)

    def drain_stores(buf, ssem):
        for b in range(NBUF):
            pltpu.make_async_copy(buf.at[b], out_hbm.at[pl.ds(base, CHUNK)],
                                  ssem).wait()

    fire_gathers(0, buf_a, gsem_a)

    def body(g2, carry):
        ga = 2 * g2
        gb = ga + 1
        drain_gathers(ga, buf_a, gsem_a)

        @pl.when(g2 > 0)
        def _():
            drain_stores(buf_b, ssem_b)

        fire_gathers(gb, buf_b, gsem_b)
        fire_stores(ga, buf_a, ssem_a)
        drain_gathers(gb, buf_b, gsem_b)
        drain_stores(buf_a, ssem_a)

        @pl.when(g2 < G2 - 1)
        def _():
            fire_gathers(ga + 2, buf_a, gsem_a)

        fire_stores(gb, buf_b, ssem_b)
        return carry

    lax.fori_loop(0, G2, body, 0)
    drain_stores(buf_b, ssem_b)


def kernel(input, weight):
    idx = jnp.swapaxes(input, 0, 1).reshape(NW, N_CHUNKS, CHUNK)
    wt = jnp.swapaxes(weight, 0, 1)
    w4 = _transpose_table(wt)
    out = _emb_lookup(idx, w4.reshape(NUM_EMB, DIM))
    out3 = out.reshape(NUM_COLS, NUM_ROWS, DIM)
    return jnp.swapaxes(out3, 0, 1)


# speedup vs baseline: 1.1432x; 1.1432x over previous
import functools

import jax
import jax.numpy as jnp
from jax import lax
from jax.experimental import pallas as pl
from jax.experimental.pallas import tpu as pltpu
from jax.experimental.pallas import tpu_sc as plsc

NUM_ROWS = 16384
NUM_COLS = 26
DIM = 32

B_TOTAL = NUM_ROWS * NUM_COLS
NUM_EMB = 1000000
NC = 2
NS = 16
NW = NC * NS
B_PER_W = B_TOTAL // NW
CHUNK = 128
N_CHUNKS = B_PER_W // CHUNK
NBUF = 4
N_GROUPS = N_CHUNKS // NBUF
G2 = N_GROUPS // 2

SB = 512
N_SLABS = NUM_EMB // SB
SLABS_PER_W = 61
TAIL0 = N_SLABS * SB
TAIL = NUM_EMB - TAIL0

_mesh = plsc.VectorSubcoreMesh(core_axis_name="c", subcore_axis_name="s")


@functools.partial(
    pl.kernel,
    out_type=jax.ShapeDtypeStruct((NUM_EMB * DIM // 128, 128), jnp.float32),
    mesh=_mesh,
    scratch_types=[
        pltpu.VMEM((DIM, SB), jnp.float32),
        pltpu.VMEM((DIM, SB), jnp.float32),
        pltpu.VMEM((DIM, TAIL), jnp.float32),
        pltpu.VMEM((SB // 4, 128), jnp.float32),
        pltpu.VMEM((SB // 4, 128), jnp.float32),
        pltpu.SemaphoreType.DMA,
        pltpu.SemaphoreType.DMA,
        pltpu.SemaphoreType.DMA,
        pltpu.SemaphoreType.DMA,
    ],
    compiler_params=pltpu.CompilerParams(use_tc_tiling_on_sc=True,
                                         needs_layout_passes=False),
)
def _transpose_table(wt_hbm, out_hbm, slab_a, slab_b, tail_slab, ot_a, ot_b,
                     isem_a, isem_b, osem_a, osem_b):
    wid = lax.axis_index("s") * NC + lax.axis_index("c")
    k0 = wid * SLABS_PER_W

    def in_slice(k):
        return wt_hbm.at[:, pl.ds(pl.multiple_of(k * SB, SB), SB)]

    def out_slice(k):
        return out_hbm.at[
            pl.ds(pl.multiple_of(k * (SB // 4), SB // 4), SB // 4)]

    def transpose_slab(width, src, ot, carry=0):
        def per_g(g, c2):
            v = g * 16 + lax.iota(jnp.int32, 16)
            row = v >> 2
            col0 = (v & 3) * DIM
            for d in range(DIM):
                plsc.store_scatter(ot, [row, col0 + d],
                                   src[d, pl.ds(g * 16, 16)])
            return c2

        return lax.fori_loop(0, width // 16, per_g, carry)

    pltpu.async_copy(in_slice(k0), slab_a, isem_a)
    pltpu.async_copy(in_slice(k0 + 1), slab_b, isem_b)

    def step(k, slab, ot, isem, osem, prefetch, drain_ot):
        pltpu.make_async_copy(in_slice(k), slab, isem).wait()

        @pl.when(drain_ot)
        def _():
            pltpu.make_async_copy(ot, out_slice(k0), osem).wait()

        transpose_slab(SB, slab, ot)
        pltpu.async_copy(ot, out_slice(k), osem)

        @pl.when(prefetch)
        def _():
            pltpu.async_copy(in_slice(k + 2), slab, isem)

    def body(p, carry):
        ka = k0 + 2 * p
        step(ka, slab_a, ot_a, isem_a, osem_a, True, p > 0)
        step(ka + 1, slab_b, ot_b, isem_b, osem_b, p < 29, p > 0)
        return carry

    lax.fori_loop(0, 30, body, 0)
    step(k0 + 60, slab_a, ot_a, isem_a, osem_a, False, True)
    pltpu.make_async_copy(ot_a, out_slice(k0), osem_a).wait()
    pltpu.make_async_copy(ot_b, out_slice(k0), osem_b).wait()

    @pl.when(wid == 0)
    def _():
        k = SLABS_PER_W * NW
        pltpu.async_copy(in_slice(k), slab_a, isem_a)
        pltpu.make_async_copy(in_slice(k), slab_a, isem_a).wait()
        transpose_slab(SB, slab_a, ot_a)
        pltpu.async_copy(ot_a, out_slice(k), osem_a).wait()
        pltpu.async_copy(wt_hbm.at[:, pl.ds(TAIL0, TAIL)], tail_slab,
                         isem_a).wait()
        transpose_slab(TAIL, tail_slab, ot_a)
        pltpu.async_copy(ot_a.at[pl.ds(0, TAIL // 4)],
                         out_hbm.at[pl.ds(TAIL0 // 4, TAIL // 4)],
                         osem_a).wait()


@functools.partial(
    pl.kernel,
    out_type=jax.ShapeDtypeStruct((B_TOTAL, DIM), jnp.float32),
    mesh=_mesh,
    scratch_types=[
        pltpu.VMEM((N_CHUNKS, CHUNK), jnp.int32),
        pltpu.VMEM((NBUF, CHUNK, DIM), jnp.float32),
        pltpu.VMEM((NBUF, CHUNK, DIM), jnp.float32),
        pltpu.SemaphoreType.DMA,
        pltpu.SemaphoreType.DMA,
        pltpu.SemaphoreType.DMA,
        pltpu.SemaphoreType.DMA,
    ],
    compiler_params=pltpu.CompilerParams(use_tc_tiling_on_sc=False),
)
def _emb_lookup(idx_hbm, table_hbm, out_hbm, idx_v, buf_a, buf_b,
                gsem_a, gsem_b, ssem_a, ssem_b):
    wid = lax.axis_index("s") * NC + lax.axis_index("c")
    base = wid * B_PER_W
    pltpu.sync_copy(idx_hbm.at[wid], idx_v)

    def fire_gathers(g, buf, gsem):
        for b in range(NBUF):
            i = g * NBUF + b
            pltpu.async_copy(table_hbm.at[idx_v.at[i]], buf.at[b], gsem)

    def drain_gathers(g, buf, gsem):
        for b in range(NBUF):
            i = g * NBUF + b
            pltpu.make_async_copy(table_hbm.at[idx_v.at[i]], buf.at[b],
                                  gsem).wait()

    def fire_stores(g, buf, ssem):
        for b in range(NBUF):
            i = g * NBUF + b
            pltpu.async_copy(buf.at[b],
                             out_hbm.at[pl.ds(base + i * CHUNK, CHUNK)], ssem)

    def drain_stores(buf, ssem):
        for b in range(NBUF):
            pltpu.make_async_copy(buf.at[b], out_hbm.at[pl.ds(base, CHUNK)],
                                  ssem).wait()

    fire_gathers(0, buf_a, gsem_a)

    def body(g2, carry):
        ga = 2 * g2
        gb = ga + 1
        drain_gathers(ga, buf_a, gsem_a)

        @pl.when(g2 > 0)
        def _():
            drain_stores(buf_b, ssem_b)

        fire_gathers(gb, buf_b, gsem_b)
        fire_stores(ga, buf_a, ssem_a)
        drain_gathers(gb, buf_b, gsem_b)
        drain_stores(buf_a, ssem_a)

        @pl.when(g2 < G2 - 1)
        def _():
            fire_gathers(ga + 2, buf_a, gsem_a)

        fire_stores(gb, buf_b, ssem_b)
        return carry

    lax.fori_loop(0, G2, body, 0)
    drain_stores(buf_b, ssem_b)


def kernel(input, weight):
    idx = jnp.swapaxes(input, 0, 1).reshape(NW, N_CHUNKS, CHUNK)
    wt = jnp.swapaxes(weight, 0, 1)
    w4 = _transpose_table(wt)
    out = _emb_lookup(idx, w4.reshape(NUM_EMB, DIM))
    out3 = out.reshape(NUM_COLS, NUM_ROWS, DIM)
    return jnp.swapaxes(out3, 0, 1)
